# initial kernel scaffold (unmeasured)
import jax
import jax.numpy as jnp
from jax import lax
from jax.experimental import pallas as pl
from jax.experimental.pallas import tpu as pltpu

N_DEV = 4
SQ = 1024
SKV = 1024
D_MODEL = 1024
H_PER = 8
DH = 128
BLK = 64
SCALE = 0.08838834764831843


def kernel(x, Wq, K_ext, V_ext, Wo):
    my_pos = lax.axis_index("i")

    k_loc = lax.dynamic_slice_in_dim(K_ext[0], my_pos * H_PER, H_PER, axis=1)
    v_loc = lax.dynamic_slice_in_dim(V_ext[0], my_pos * H_PER, H_PER, axis=1)
    k2 = k_loc.reshape(SKV, H_PER * DH)
    v2 = v_loc.reshape(SKV, H_PER * DH)
    x2 = x[0]

    def body(x_ref, wq_ref, k_ref, v_ref, wo_ref, out_ref,
             ctx_ref, comm_ref, send_sems, recv_sems):
        left = (my_pos - 1) % N_DEV
        right = (my_pos + 1) % N_DEV

        barrier_sem = pltpu.get_barrier_semaphore()
        for nbr in [left, right]:
            pl.semaphore_signal(
                barrier_sem, inc=1,
                device_id=(nbr,), device_id_type=pl.DeviceIdType.MESH,
            )
        pl.semaphore_wait(barrier_sem, 2)

        q = jnp.dot(x_ref[:, :], wq_ref[:, :],
                    preferred_element_type=jnp.float32)

        rows = lax.broadcasted_iota(jnp.int32, (SQ, SKV), 0) // BLK
        cols = lax.broadcasted_iota(jnp.int32, (SQ, SKV), 1) // BLK
        mask = (rows == cols) | (cols == 0) | ((rows + cols) % 3 == 0)

        for h in range(H_PER):
            sl = pl.ds(h * DH, DH)
            qh = q[:, h * DH:(h + 1) * DH]
            kh = k_ref[:, sl]
            scores = lax.dot_general(
                qh, kh, (((1,), (1,)), ((), ())),
                preferred_element_type=jnp.float32,
            ) * SCALE
            scores = jnp.where(mask, scores, -1e9)
            m = jnp.max(scores, axis=-1, keepdims=True)
            w = jnp.exp(scores - m)
            w = w / jnp.sum(w, axis=-1, keepdims=True)
            ctx_ref[:, sl] = jnp.dot(w, v_ref[:, sl],
                                     preferred_element_type=jnp.float32)

        partial = jnp.dot(ctx_ref[:, :], wo_ref[:, :],
                          preferred_element_type=jnp.float32)
        out_ref[:, :] = partial
        comm_ref[0, :, :] = partial

        for h in range(N_DEV - 1):
            send_slot = h % 2
            recv_slot = (h + 1) % 2
            rdma = pltpu.make_async_remote_copy(
                src_ref=comm_ref.at[send_slot],
                dst_ref=comm_ref.at[recv_slot],
                send_sem=send_sems.at[send_slot],
                recv_sem=recv_sems.at[recv_slot],
                device_id=(right,),
                device_id_type=pl.DeviceIdType.MESH,
            )
            rdma.start()
            rdma.wait()
            out_ref[:, :] += comm_ref[recv_slot, :, :]

    out = pl.pallas_call(
        body,
        out_shape=jax.ShapeDtypeStruct((SQ, D_MODEL), jnp.float32),
        in_specs=[pl.BlockSpec(memory_space=pltpu.VMEM)] * 5,
        out_specs=pl.BlockSpec(memory_space=pltpu.VMEM),
        scratch_shapes=[
            pltpu.VMEM((SQ, H_PER * DH), jnp.float32),
            pltpu.VMEM((2, SQ, D_MODEL), jnp.float32),
            pltpu.SemaphoreType.DMA((2,)),
            pltpu.SemaphoreType.DMA((2,)),
        ],
        compiler_params=pltpu.CompilerParams(collective_id=0),
    )(x2, Wq, k2, v2, Wo)

    return out.reshape(1, SQ, D_MODEL)


# baseline (device time: 183227 ns/iter reference)
import jax
import jax.numpy as jnp
from jax import lax
from jax.experimental import pallas as pl
from jax.experimental.pallas import tpu as pltpu

N_DEV = 4
SQ = 1024
SKV = 1024
D_MODEL = 1024
H_PER = 8
DH = 128
BLK = 64
SCALE = 0.08838834764831843


def kernel(x, Wq, K_ext, V_ext, Wo):
    my_pos = lax.axis_index("i")

    k_loc = lax.dynamic_slice_in_dim(K_ext[0], my_pos * H_PER, H_PER, axis=1)
    v_loc = lax.dynamic_slice_in_dim(V_ext[0], my_pos * H_PER, H_PER, axis=1)
    k2 = k_loc.reshape(SKV, H_PER * DH)
    v2 = v_loc.reshape(SKV, H_PER * DH)
    x2 = x[0]

    def body(x_ref, wq_ref, k_ref, v_ref, wo_ref, out_ref,
             ctx_ref, comm_ref, send_sems, recv_sems):
        pos = lax.axis_index("i")
        left = (pos - 1) % N_DEV
        right = (pos + 1) % N_DEV

        barrier_sem = pltpu.get_barrier_semaphore()
        for nbr in [left, right]:
            pl.semaphore_signal(
                barrier_sem, inc=1,
                device_id=(nbr,), device_id_type=pl.DeviceIdType.MESH,
            )
        pl.semaphore_wait(barrier_sem, 2)

        q = jnp.dot(x_ref[:, :], wq_ref[:, :],
                    preferred_element_type=jnp.float32)

        rows = lax.broadcasted_iota(jnp.int32, (SQ, SKV), 0) // BLK
        cols = lax.broadcasted_iota(jnp.int32, (SQ, SKV), 1) // BLK
        mask = (rows == cols) | (cols == 0) | ((rows + cols) % 3 == 0)

        for h in range(H_PER):
            sl = pl.ds(h * DH, DH)
            qh = q[:, h * DH:(h + 1) * DH]
            kh = k_ref[:, sl]
            scores = lax.dot_general(
                qh, kh, (((1,), (1,)), ((), ())),
                preferred_element_type=jnp.float32,
            ) * SCALE
            scores = jnp.where(mask, scores, -1e9)
            m = jnp.max(scores, axis=-1, keepdims=True)
            w = jnp.exp(scores - m)
            w = w / jnp.sum(w, axis=-1, keepdims=True)
            ctx_ref[:, sl] = jnp.dot(w, v_ref[:, sl],
                                     preferred_element_type=jnp.float32)

        partial = jnp.dot(ctx_ref[:, :], wo_ref[:, :],
                          preferred_element_type=jnp.float32)
        out_ref[:, :] = partial
        comm_ref[0, :, :] = partial

        for h in range(N_DEV - 1):
            send_slot = h % 2
            recv_slot = (h + 1) % 2
            rdma = pltpu.make_async_remote_copy(
                src_ref=comm_ref.at[send_slot],
                dst_ref=comm_ref.at[recv_slot],
                send_sem=send_sems.at[send_slot],
                recv_sem=recv_sems.at[recv_slot],
                device_id=(right,),
                device_id_type=pl.DeviceIdType.MESH,
            )
            rdma.start()
            rdma.wait()
            out_ref[:, :] += comm_ref[recv_slot, :, :]

    out = pl.pallas_call(
        body,
        out_shape=jax.ShapeDtypeStruct((SQ, D_MODEL), jnp.float32),
        in_specs=[pl.BlockSpec(memory_space=pltpu.VMEM)] * 5,
        out_specs=pl.BlockSpec(memory_space=pltpu.VMEM),
        scratch_shapes=[
            pltpu.VMEM((SQ, H_PER * DH), jnp.float32),
            pltpu.VMEM((2, SQ, D_MODEL), jnp.float32),
            pltpu.SemaphoreType.DMA((2,)),
            pltpu.SemaphoreType.DMA((2,)),
        ],
        compiler_params=pltpu.CompilerParams(collective_id=0),
    )(x2, Wq, k2, v2, Wo)

    return out.reshape(1, SQ, D_MODEL)


# device time: 85285 ns/iter; 2.1484x vs baseline; 2.1484x over previous
import jax
import jax.numpy as jnp
from jax import lax
from jax.experimental import pallas as pl
from jax.experimental.pallas import tpu as pltpu

N_DEV = 4
SQ = 1024
SKV = 1024
D_MODEL = 1024
H_PER = 8
DH = 128
BLK = 64
SCALE = 0.08838834764831843


def kernel(x, Wq, K_ext, V_ext, Wo):
    my_pos = lax.axis_index("i")

    k_loc = lax.dynamic_slice_in_dim(K_ext[0], my_pos * H_PER, H_PER, axis=1)
    v_loc = lax.dynamic_slice_in_dim(V_ext[0], my_pos * H_PER, H_PER, axis=1)
    k2 = k_loc.reshape(SKV, H_PER * DH)
    v2 = v_loc.reshape(SKV, H_PER * DH)
    x2 = x[0]

    def body(x_ref, wq_ref, k_ref, v_ref, wo_ref, out_ref,
             ctx_ref, stag_cw, stag_ccw,
             send_cw, recv_cw, send_ccw, recv_ccw):
        pos = lax.axis_index("i")
        left = (pos - 1) % N_DEV
        right = (pos + 1) % N_DEV

        barrier_sem = pltpu.get_barrier_semaphore()
        for nbr in [left, right]:
            pl.semaphore_signal(
                barrier_sem, inc=1,
                device_id=(nbr,), device_id_type=pl.DeviceIdType.MESH,
            )
        pl.semaphore_wait(barrier_sem, 2)

        q = jnp.dot(x_ref[:, :], wq_ref[:, :],
                    preferred_element_type=jnp.float32)

        rows = lax.broadcasted_iota(jnp.int32, (SQ, SKV), 0) // BLK
        cols = lax.broadcasted_iota(jnp.int32, (SQ, SKV), 1) // BLK
        mask = (rows == cols) | (cols == 0) | ((rows + cols) % 3 == 0)

        for h in range(H_PER):
            sl = pl.ds(h * DH, DH)
            qh = q[:, h * DH:(h + 1) * DH]
            kh = k_ref[:, sl]
            scores = lax.dot_general(
                qh, kh, (((1,), (1,)), ((), ())),
                preferred_element_type=jnp.float32,
            ) * SCALE
            scores = jnp.where(mask, scores, -1e9)
            m = jnp.max(scores, axis=-1, keepdims=True)
            w = jnp.exp(scores - m)
            w = w / jnp.sum(w, axis=-1, keepdims=True)
            ctx_ref[:, sl] = jnp.dot(w, v_ref[:, sl],
                                     preferred_element_type=jnp.float32)

        out_ref[:, :] = jnp.dot(ctx_ref[:, :], wo_ref[:, :],
                                preferred_element_type=jnp.float32)

        RCH = SQ // N_DEV

        def rows(c):
            return pl.ds(c * RCH, RCH)

        CL = pl.ds(0, D_MODEL // 2)
        CR = pl.ds(D_MODEL // 2, D_MODEL // 2)

        for s in range(N_DEV - 1):
            cw = pltpu.make_async_remote_copy(
                src_ref=out_ref.at[rows((pos - s) % N_DEV), CL],
                dst_ref=stag_cw.at[s],
                send_sem=send_cw.at[s], recv_sem=recv_cw.at[s],
                device_id=(right,), device_id_type=pl.DeviceIdType.MESH,
            )
            ccw = pltpu.make_async_remote_copy(
                src_ref=out_ref.at[rows((pos + s) % N_DEV), CR],
                dst_ref=stag_ccw.at[s],
                send_sem=send_ccw.at[s], recv_sem=recv_ccw.at[s],
                device_id=(left,), device_id_type=pl.DeviceIdType.MESH,
            )
            cw.start()
            ccw.start()
            cw.wait()
            ccw.wait()
            out_ref[rows((pos - 1 - s) % N_DEV), CL] += stag_cw[s]
            out_ref[rows((pos + 1 + s) % N_DEV), CR] += stag_ccw[s]

        for t in range(N_DEV - 1):
            g_cw = (pos + 1 - t) % N_DEV
            g_ccw = (pos - 1 + t) % N_DEV
            cw = pltpu.make_async_remote_copy(
                src_ref=out_ref.at[rows(g_cw), CL],
                dst_ref=out_ref.at[rows(g_cw), CL],
                send_sem=send_cw.at[3 + t], recv_sem=recv_cw.at[3 + t],
                device_id=(right,), device_id_type=pl.DeviceIdType.MESH,
            )
            ccw = pltpu.make_async_remote_copy(
                src_ref=out_ref.at[rows(g_ccw), CR],
                dst_ref=out_ref.at[rows(g_ccw), CR],
                send_sem=send_ccw.at[3 + t], recv_sem=recv_ccw.at[3 + t],
                device_id=(left,), device_id_type=pl.DeviceIdType.MESH,
            )
            cw.start()
            ccw.start()
            cw.wait()
            ccw.wait()

    out = pl.pallas_call(
        body,
        out_shape=jax.ShapeDtypeStruct((SQ, D_MODEL), jnp.float32),
        in_specs=[pl.BlockSpec(memory_space=pltpu.VMEM)] * 5,
        out_specs=pl.BlockSpec(memory_space=pltpu.VMEM),
        scratch_shapes=[
            pltpu.VMEM((SQ, H_PER * DH), jnp.float32),
            pltpu.VMEM((3, SQ // N_DEV, D_MODEL // 2), jnp.float32),
            pltpu.VMEM((3, SQ // N_DEV, D_MODEL // 2), jnp.float32),
            pltpu.SemaphoreType.DMA((6,)),
            pltpu.SemaphoreType.DMA((6,)),
            pltpu.SemaphoreType.DMA((6,)),
            pltpu.SemaphoreType.DMA((6,)),
        ],
        compiler_params=pltpu.CompilerParams(collective_id=0),
    )(x2, Wq, k2, v2, Wo)

    return out.reshape(1, SQ, D_MODEL)


# device time: 75235 ns/iter; 2.4354x vs baseline; 1.1336x over previous
import jax
import jax.numpy as jnp
from jax import lax
from jax.experimental import pallas as pl
from jax.experimental.pallas import tpu as pltpu

N_DEV = 4
SQ = 1024
SKV = 1024
D_MODEL = 1024
H_PER = 8
DH = 128
BLK = 64
SCALE = 0.08838834764831843


def kernel(x, Wq, K_ext, V_ext, Wo):
    my_pos = lax.axis_index("i")

    k_loc = lax.dynamic_slice_in_dim(K_ext[0], my_pos * H_PER, H_PER, axis=1)
    v_loc = lax.dynamic_slice_in_dim(V_ext[0], my_pos * H_PER, H_PER, axis=1)
    k2 = k_loc.reshape(SKV, H_PER * DH)
    v2 = v_loc.reshape(SKV, H_PER * DH)
    x2 = x[0]

    def body(x_ref, wq_ref, k_ref, v_ref, wo_ref, out_ref,
             ctx_ref, stag_cw, stag_ccw,
             send_cw, recv_cw, send_ccw, recv_ccw):
        pos = lax.axis_index("i")
        left = (pos - 1) % N_DEV
        right = (pos + 1) % N_DEV

        barrier_sem = pltpu.get_barrier_semaphore()
        for nbr in [left, right]:
            pl.semaphore_signal(
                barrier_sem, inc=1,
                device_id=(nbr,), device_id_type=pl.DeviceIdType.MESH,
            )
        pl.semaphore_wait(barrier_sem, 2)

        RCH = SQ // N_DEV

        def rows(c):
            return pl.ds(c * RCH, RCH)

        CL = pl.ds(0, D_MODEL // 2)
        CR = pl.ds(D_MODEL // 2, D_MODEL // 2)

        def compute_chunk(c):
            q = jnp.dot(x_ref[rows(c), :], wq_ref[:, :],
                        preferred_element_type=jnp.float32)
            qb = (lax.broadcasted_iota(jnp.int32, (RCH, SKV), 0)
                  + c * RCH) // BLK
            kb = lax.broadcasted_iota(jnp.int32, (RCH, SKV), 1) // BLK
            mask = (qb == kb) | (kb == 0) | ((qb + kb) % 3 == 0)
            for h in range(H_PER):
                sl = pl.ds(h * DH, DH)
                qh = q[:, h * DH:(h + 1) * DH]
                kh = k_ref[:, sl]
                scores = lax.dot_general(
                    qh, kh, (((1,), (1,)), ((), ())),
                    preferred_element_type=jnp.float32,
                ) * SCALE
                scores = jnp.where(mask, scores, -1e9)
                m = jnp.max(scores, axis=-1, keepdims=True)
                w = jnp.exp(scores - m)
                w = w / jnp.sum(w, axis=-1, keepdims=True)
                ctx_ref[:, sl] = jnp.dot(w, v_ref[:, sl],
                                         preferred_element_type=jnp.float32)
            out_ref[rows(c), :] = jnp.dot(
                ctx_ref[:, :], wo_ref[:, :],
                preferred_element_type=jnp.float32)

        def cw_rdma(s):
            return pltpu.make_async_remote_copy(
                src_ref=out_ref.at[rows((pos - s) % N_DEV), CL],
                dst_ref=stag_cw.at[s],
                send_sem=send_cw.at[s], recv_sem=recv_cw.at[s],
                device_id=(right,), device_id_type=pl.DeviceIdType.MESH,
            )

        def ccw_rdma(s):
            return pltpu.make_async_remote_copy(
                src_ref=out_ref.at[rows((pos + s) % N_DEV), CR],
                dst_ref=stag_ccw.at[s],
                send_sem=send_ccw.at[s], recv_sem=recv_ccw.at[s],
                device_id=(left,), device_id_type=pl.DeviceIdType.MESH,
            )

        compute_chunk(pos)
        cw_rdma(0).start()
        ccw_rdma(0).start()

        compute_chunk((pos + 1) % N_DEV)
        ccw_rdma(0).wait()
        out_ref[rows((pos + 1) % N_DEV), CR] += stag_ccw[0]
        ccw_rdma(1).start()

        compute_chunk((pos + 3) % N_DEV)
        cw_rdma(0).wait()
        out_ref[rows((pos - 1) % N_DEV), CL] += stag_cw[0]
        cw_rdma(1).start()

        compute_chunk((pos + 2) % N_DEV)
        ccw_rdma(1).wait()
        out_ref[rows((pos + 2) % N_DEV), CR] += stag_ccw[1]
        ccw_rdma(2).start()

        cw_rdma(1).wait()
        out_ref[rows((pos - 2) % N_DEV), CL] += stag_cw[1]
        cw_rdma(2).start()

        ccw_rdma(2).wait()
        out_ref[rows((pos + 3) % N_DEV), CR] += stag_ccw[2]
        cw_rdma(2).wait()
        out_ref[rows((pos - 3) % N_DEV), CL] += stag_cw[2]

        for t in range(N_DEV - 1):
            g_cw = (pos + 1 - t) % N_DEV
            g_ccw = (pos - 1 + t) % N_DEV
            cw = pltpu.make_async_remote_copy(
                src_ref=out_ref.at[rows(g_cw), CL],
                dst_ref=out_ref.at[rows(g_cw), CL],
                send_sem=send_cw.at[3 + t], recv_sem=recv_cw.at[3 + t],
                device_id=(right,), device_id_type=pl.DeviceIdType.MESH,
            )
            ccw = pltpu.make_async_remote_copy(
                src_ref=out_ref.at[rows(g_ccw), CR],
                dst_ref=out_ref.at[rows(g_ccw), CR],
                send_sem=send_ccw.at[3 + t], recv_sem=recv_ccw.at[3 + t],
                device_id=(left,), device_id_type=pl.DeviceIdType.MESH,
            )
            cw.start()
            ccw.start()
            cw.wait()
            ccw.wait()

    out = pl.pallas_call(
        body,
        out_shape=jax.ShapeDtypeStruct((SQ, D_MODEL), jnp.float32),
        in_specs=[pl.BlockSpec(memory_space=pltpu.VMEM)] * 5,
        out_specs=pl.BlockSpec(memory_space=pltpu.VMEM),
        scratch_shapes=[
            pltpu.VMEM((SQ // N_DEV, H_PER * DH), jnp.float32),
            pltpu.VMEM((3, SQ // N_DEV, D_MODEL // 2), jnp.float32),
            pltpu.VMEM((3, SQ // N_DEV, D_MODEL // 2), jnp.float32),
            pltpu.SemaphoreType.DMA((6,)),
            pltpu.SemaphoreType.DMA((6,)),
            pltpu.SemaphoreType.DMA((6,)),
            pltpu.SemaphoreType.DMA((6,)),
        ],
        compiler_params=pltpu.CompilerParams(collective_id=0),
    )(x2, Wq, k2, v2, Wo)

    return out.reshape(1, SQ, D_MODEL)


# device time: 68440 ns/iter; 2.6772x vs baseline; 1.0993x over previous
import jax
import jax.numpy as jnp
from jax import lax
from jax.experimental import pallas as pl
from jax.experimental.pallas import tpu as pltpu

N_DEV = 4
SQ = 1024
SKV = 1024
D_MODEL = 1024
H_PER = 8
DH = 128
BLK = 64
SCALE = 0.08838834764831843
BF = jnp.bfloat16


def kernel(x, Wq, K_ext, V_ext, Wo):
    my_pos = lax.axis_index("i")

    k_loc = lax.dynamic_slice_in_dim(K_ext[0], my_pos * H_PER, H_PER, axis=1)
    v_loc = lax.dynamic_slice_in_dim(V_ext[0], my_pos * H_PER, H_PER, axis=1)
    k2 = k_loc.reshape(SKV, H_PER * DH).astype(BF)
    v2 = v_loc.reshape(SKV, H_PER * DH).astype(BF)
    x2 = x[0].astype(BF)
    wq2 = Wq.astype(BF)
    wo2 = Wo.astype(BF)

    def body(x_ref, wq_ref, k_ref, v_ref, wo_ref, out_ref,
             ctx_ref, pbuf, stag_cw, stag_ccw,
             send_cw, recv_cw, send_ccw, recv_ccw):
        pos = lax.axis_index("i")
        left = (pos - 1) % N_DEV
        right = (pos + 1) % N_DEV

        barrier_sem = pltpu.get_barrier_semaphore()
        for nbr in [left, right]:
            pl.semaphore_signal(
                barrier_sem, inc=1,
                device_id=(nbr,), device_id_type=pl.DeviceIdType.MESH,
            )
        pl.semaphore_wait(barrier_sem, 2)

        RCH = SQ // N_DEV

        def rows(c):
            return pl.ds(c * RCH, RCH)

        CL = pl.ds(0, D_MODEL // 2)
        CR = pl.ds(D_MODEL // 2, D_MODEL // 2)

        def compute_chunk(c):
            q = jnp.dot(x_ref[rows(c), :], wq_ref[:, :],
                        preferred_element_type=jnp.float32)
            qb = (lax.broadcasted_iota(jnp.int32, (RCH, SKV), 0)
                  + c * RCH) // BLK
            kb = lax.broadcasted_iota(jnp.int32, (RCH, SKV), 1) // BLK
            mask = (qb == kb) | (kb == 0) | ((qb + kb) % 3 == 0)
            for h in range(H_PER):
                sl = pl.ds(h * DH, DH)
                qh = q[:, h * DH:(h + 1) * DH].astype(BF)
                kh = k_ref[:, sl]
                scores = lax.dot_general(
                    qh, kh, (((1,), (1,)), ((), ())),
                    preferred_element_type=jnp.float32,
                ) * SCALE
                scores = jnp.where(mask, scores, -1e9)
                m = jnp.max(scores, axis=-1, keepdims=True)
                w = jnp.exp(scores - m)
                w = (w / jnp.sum(w, axis=-1, keepdims=True)).astype(BF)
                ctx_ref[:, sl] = jnp.dot(
                    w, v_ref[:, sl],
                    preferred_element_type=jnp.float32).astype(BF)
            pbuf[rows(c), :] = jnp.dot(
                ctx_ref[:, :], wo_ref[:, :],
                preferred_element_type=jnp.float32).astype(BF)

        def cw_rdma(s):
            return pltpu.make_async_remote_copy(
                src_ref=pbuf.at[rows((pos - s) % N_DEV), CL],
                dst_ref=stag_cw.at[s],
                send_sem=send_cw.at[s], recv_sem=recv_cw.at[s],
                device_id=(right,), device_id_type=pl.DeviceIdType.MESH,
            )

        def ccw_rdma(s):
            return pltpu.make_async_remote_copy(
                src_ref=pbuf.at[rows((pos + s) % N_DEV), CR],
                dst_ref=stag_ccw.at[s],
                send_sem=send_ccw.at[s], recv_sem=recv_ccw.at[s],
                device_id=(left,), device_id_type=pl.DeviceIdType.MESH,
            )

        def accum(c, half, stag, s):
            cur = pbuf[rows(c), half].astype(jnp.float32)
            inc = stag[s].astype(jnp.float32)
            pbuf[rows(c), half] = (cur + inc).astype(BF)

        compute_chunk(pos)
        cw_rdma(0).start()
        ccw_rdma(0).start()

        compute_chunk((pos + 1) % N_DEV)
        ccw_rdma(0).wait()
        accum((pos + 1) % N_DEV, CR, stag_ccw, 0)
        ccw_rdma(1).start()

        compute_chunk((pos + 3) % N_DEV)
        cw_rdma(0).wait()
        accum((pos - 1) % N_DEV, CL, stag_cw, 0)
        cw_rdma(1).start()

        compute_chunk((pos + 2) % N_DEV)
        ccw_rdma(1).wait()
        accum((pos + 2) % N_DEV, CR, stag_ccw, 1)
        ccw_rdma(2).start()

        cw_rdma(1).wait()
        accum((pos - 2) % N_DEV, CL, stag_cw, 1)
        cw_rdma(2).start()

        ccw_rdma(2).wait()
        accum((pos + 3) % N_DEV, CR, stag_ccw, 2)
        cw_rdma(2).wait()
        accum((pos - 3) % N_DEV, CL, stag_cw, 2)

        for t in range(N_DEV - 1):
            g_cw = (pos + 1 - t) % N_DEV
            g_ccw = (pos - 1 + t) % N_DEV
            cw = pltpu.make_async_remote_copy(
                src_ref=pbuf.at[rows(g_cw), CL],
                dst_ref=pbuf.at[rows(g_cw), CL],
                send_sem=send_cw.at[3 + t], recv_sem=recv_cw.at[3 + t],
                device_id=(right,), device_id_type=pl.DeviceIdType.MESH,
            )
            ccw = pltpu.make_async_remote_copy(
                src_ref=pbuf.at[rows(g_ccw), CR],
                dst_ref=pbuf.at[rows(g_ccw), CR],
                send_sem=send_ccw.at[3 + t], recv_sem=recv_ccw.at[3 + t],
                device_id=(left,), device_id_type=pl.DeviceIdType.MESH,
            )
            cw.start()
            ccw.start()
            cw.wait()
            ccw.wait()

        out_ref[:, :] = pbuf[:, :].astype(jnp.float32)

    out = pl.pallas_call(
        body,
        out_shape=jax.ShapeDtypeStruct((SQ, D_MODEL), jnp.float32),
        in_specs=[pl.BlockSpec(memory_space=pltpu.VMEM)] * 5,
        out_specs=pl.BlockSpec(memory_space=pltpu.VMEM),
        scratch_shapes=[
            pltpu.VMEM((SQ // N_DEV, H_PER * DH), BF),
            pltpu.VMEM((SQ, D_MODEL), BF),
            pltpu.VMEM((3, SQ // N_DEV, D_MODEL // 2), BF),
            pltpu.VMEM((3, SQ // N_DEV, D_MODEL // 2), BF),
            pltpu.SemaphoreType.DMA((6,)),
            pltpu.SemaphoreType.DMA((6,)),
            pltpu.SemaphoreType.DMA((6,)),
            pltpu.SemaphoreType.DMA((6,)),
        ],
        compiler_params=pltpu.CompilerParams(collective_id=0),
    )(x2, wq2, k2, v2, wo2)

    return out.reshape(1, SQ, D_MODEL)


# device time: 58381 ns/iter; 3.1385x vs baseline; 1.1723x over previous
import jax
import jax.numpy as jnp
from jax import lax
from jax.experimental import pallas as pl
from jax.experimental.pallas import tpu as pltpu

N_DEV = 4
SQ = 1024
SKV = 1024
D_MODEL = 1024
H_PER = 8
DH = 128
BLK = 64
SCALE = 0.08838834764831843
BF = jnp.bfloat16


def kernel(x, Wq, K_ext, V_ext, Wo):
    my_pos = lax.axis_index("i")

    k_loc = lax.dynamic_slice_in_dim(K_ext[0], my_pos * H_PER, H_PER, axis=1)
    v_loc = lax.dynamic_slice_in_dim(V_ext[0], my_pos * H_PER, H_PER, axis=1)
    k2 = k_loc.reshape(SKV, H_PER * DH).astype(BF)
    v2 = v_loc.reshape(SKV, H_PER * DH).astype(BF)
    x2 = x[0].astype(BF)
    wq2 = Wq.astype(BF)
    wo2 = Wo.astype(BF)

    def body(x_ref, wq_ref, k_ref, v_ref, wo_ref, out_ref,
             ctx_ref, pbuf, stag_cw, stag_ccw,
             send_cw, recv_cw, send_ccw, recv_ccw):
        pos = lax.axis_index("i")
        left = (pos - 1) % N_DEV
        right = (pos + 1) % N_DEV

        barrier_sem = pltpu.get_barrier_semaphore()
        for nbr in [left, right]:
            pl.semaphore_signal(
                barrier_sem, inc=1,
                device_id=(nbr,), device_id_type=pl.DeviceIdType.MESH,
            )
        pl.semaphore_wait(barrier_sem, 2)

        RCH = SQ // N_DEV

        def rows(c):
            return pl.ds(c * RCH, RCH)

        CL = pl.ds(0, D_MODEL // 2)
        CR = pl.ds(D_MODEL // 2, D_MODEL // 2)

        def compute_chunk(c):
            q = jnp.dot(x_ref[rows(c), :], wq_ref[:, :],
                        preferred_element_type=jnp.float32)
            qb = (lax.broadcasted_iota(jnp.int32, (RCH, SKV), 0)
                  + c * RCH) // BLK
            kb = lax.broadcasted_iota(jnp.int32, (RCH, SKV), 1) // BLK
            mask = (qb == kb) | (kb == 0) | ((qb + kb) % 3 == 0)
            maskf = mask.astype(jnp.float32)
            for h in range(H_PER):
                sl = pl.ds(h * DH, DH)
                qh = q[:, h * DH:(h + 1) * DH].astype(BF)
                kh = k_ref[:, sl]
                scores = lax.dot_general(
                    qh, kh, (((1,), (1,)), ((), ())),
                    preferred_element_type=jnp.float32,
                )
                w = jnp.exp(scores * SCALE) * maskf
                wsum = jnp.sum(w, axis=-1, keepdims=True)
                ctx = jnp.dot(w.astype(BF), v_ref[:, sl],
                              preferred_element_type=jnp.float32)
                ctx_ref[:, sl] = (ctx / wsum).astype(BF)
            pbuf[rows(c), :] = jnp.dot(
                ctx_ref[:, :], wo_ref[:, :],
                preferred_element_type=jnp.float32).astype(BF)

        def cw_rdma(s):
            return pltpu.make_async_remote_copy(
                src_ref=pbuf.at[rows((pos - s) % N_DEV), CL],
                dst_ref=stag_cw.at[s],
                send_sem=send_cw.at[s], recv_sem=recv_cw.at[s],
                device_id=(right,), device_id_type=pl.DeviceIdType.MESH,
            )

        def ccw_rdma(s):
            return pltpu.make_async_remote_copy(
                src_ref=pbuf.at[rows((pos + s) % N_DEV), CR],
                dst_ref=stag_ccw.at[s],
                send_sem=send_ccw.at[s], recv_sem=recv_ccw.at[s],
                device_id=(left,), device_id_type=pl.DeviceIdType.MESH,
            )

        def accum(c, half, stag, s):
            cur = pbuf[rows(c), half].astype(jnp.float32)
            inc = stag[s].astype(jnp.float32)
            pbuf[rows(c), half] = (cur + inc).astype(BF)

        compute_chunk(pos)
        cw_rdma(0).start()
        ccw_rdma(0).start()

        compute_chunk((pos + 1) % N_DEV)
        ccw_rdma(0).wait()
        accum((pos + 1) % N_DEV, CR, stag_ccw, 0)
        ccw_rdma(1).start()

        compute_chunk((pos + 3) % N_DEV)
        cw_rdma(0).wait()
        accum((pos - 1) % N_DEV, CL, stag_cw, 0)
        cw_rdma(1).start()

        compute_chunk((pos + 2) % N_DEV)
        ccw_rdma(1).wait()
        accum((pos + 2) % N_DEV, CR, stag_ccw, 1)
        ccw_rdma(2).start()

        cw_rdma(1).wait()
        accum((pos - 2) % N_DEV, CL, stag_cw, 1)
        cw_rdma(2).start()

        ccw_rdma(2).wait()
        accum((pos + 3) % N_DEV, CR, stag_ccw, 2)
        cw_rdma(2).wait()
        accum((pos - 3) % N_DEV, CL, stag_cw, 2)

        def ag_cw(t):
            g = (pos + 1 - t) % N_DEV
            return pltpu.make_async_remote_copy(
                src_ref=pbuf.at[rows(g), CL],
                dst_ref=pbuf.at[rows(g), CL],
                send_sem=send_cw.at[3 + t], recv_sem=recv_cw.at[3 + t],
                device_id=(right,), device_id_type=pl.DeviceIdType.MESH,
            )

        def ag_ccw(t):
            g = (pos - 1 + t) % N_DEV
            return pltpu.make_async_remote_copy(
                src_ref=pbuf.at[rows(g), CR],
                dst_ref=pbuf.at[rows(g), CR],
                send_sem=send_ccw.at[3 + t], recv_sem=recv_ccw.at[3 + t],
                device_id=(left,), device_id_type=pl.DeviceIdType.MESH,
            )

        def conv(c, half):
            out_ref[rows(c), half] = pbuf[rows(c), half].astype(jnp.float32)

        ag_cw(0).start()
        ag_ccw(0).start()
        conv((pos + 1) % N_DEV, CL)
        conv((pos - 1) % N_DEV, CR)

        ag_cw(0).wait()
        ag_cw(1).start()
        conv(pos, CL)
        ag_ccw(0).wait()
        ag_ccw(1).start()
        conv(pos, CR)

        ag_cw(1).wait()
        ag_cw(2).start()
        conv((pos - 1) % N_DEV, CL)
        ag_ccw(1).wait()
        ag_ccw(2).start()
        conv((pos + 1) % N_DEV, CR)

        ag_cw(2).wait()
        conv((pos - 2) % N_DEV, CL)
        ag_ccw(2).wait()
        conv((pos + 2) % N_DEV, CR)

    out = pl.pallas_call(
        body,
        out_shape=jax.ShapeDtypeStruct((SQ, D_MODEL), jnp.float32),
        in_specs=[pl.BlockSpec(memory_space=pltpu.VMEM)] * 5,
        out_specs=pl.BlockSpec(memory_space=pltpu.VMEM),
        scratch_shapes=[
            pltpu.VMEM((SQ // N_DEV, H_PER * DH), BF),
            pltpu.VMEM((SQ, D_MODEL), BF),
            pltpu.VMEM((3, SQ // N_DEV, D_MODEL // 2), BF),
            pltpu.VMEM((3, SQ // N_DEV, D_MODEL // 2), BF),
            pltpu.SemaphoreType.DMA((6,)),
            pltpu.SemaphoreType.DMA((6,)),
            pltpu.SemaphoreType.DMA((6,)),
            pltpu.SemaphoreType.DMA((6,)),
        ],
        compiler_params=pltpu.CompilerParams(collective_id=0),
    )(x2, wq2, k2, v2, wo2)

    return out.reshape(1, SQ, D_MODEL)


# device time: 54844 ns/iter; 3.3409x vs baseline; 1.0645x over previous
import jax
import jax.numpy as jnp
from jax import lax
from jax.experimental import pallas as pl
from jax.experimental.pallas import tpu as pltpu

N_DEV = 4
SQ = 1024
SKV = 1024
D_MODEL = 1024
H_PER = 8
DH = 128
BLK = 64
SCALE = 0.08838834764831843
BF = jnp.bfloat16


def kernel(x, Wq, K_ext, V_ext, Wo):
    x2 = x[0].astype(BF)
    wq2 = Wq.astype(BF)
    wo2 = Wo.astype(BF)

    def body(x_ref, wq_ref, kext_ref, vext_ref, wo_ref, out_ref,
             ctx_ref, pbuf, stag_cw, stag_ccw, kf32, vf32, kbf, vbf,
             send_cw, recv_cw, send_ccw, recv_ccw, kv_sems):
        pos = lax.axis_index("i")
        left = (pos - 1) % N_DEV
        right = (pos + 1) % N_DEV

        kdma = pltpu.make_async_copy(
            kext_ref.at[0, :, pl.ds(pos * H_PER, H_PER), :], kf32,
            kv_sems.at[0])
        vdma = pltpu.make_async_copy(
            vext_ref.at[0, :, pl.ds(pos * H_PER, H_PER), :], vf32,
            kv_sems.at[1])
        kdma.start()
        vdma.start()

        barrier_sem = pltpu.get_barrier_semaphore()
        for nbr in [left, right]:
            pl.semaphore_signal(
                barrier_sem, inc=1,
                device_id=(nbr,), device_id_type=pl.DeviceIdType.MESH,
            )
        pl.semaphore_wait(barrier_sem, 2)

        kdma.wait()
        kbf[:, :] = kf32[:, :, :].reshape(SKV, H_PER * DH).astype(BF)
        vdma.wait()
        vbf[:, :] = vf32[:, :, :].reshape(SKV, H_PER * DH).astype(BF)
        k_ref = kbf
        v_ref = vbf

        RCH = SQ // N_DEV

        def rows(c):
            return pl.ds(c * RCH, RCH)

        CL = pl.ds(0, D_MODEL // 2)
        CR = pl.ds(D_MODEL // 2, D_MODEL // 2)

        def compute_chunk(c):
            q = jnp.dot(x_ref[rows(c), :], wq_ref[:, :],
                        preferred_element_type=jnp.float32)
            qb = (lax.broadcasted_iota(jnp.int32, (RCH, SKV), 0)
                  + c * RCH) // BLK
            kb = lax.broadcasted_iota(jnp.int32, (RCH, SKV), 1) // BLK
            mask = (qb == kb) | (kb == 0) | ((qb + kb) % 3 == 0)
            maskf = mask.astype(jnp.float32)
            for h in range(H_PER):
                sl = pl.ds(h * DH, DH)
                qh = q[:, h * DH:(h + 1) * DH].astype(BF)
                kh = k_ref[:, sl]
                scores = lax.dot_general(
                    qh, kh, (((1,), (1,)), ((), ())),
                    preferred_element_type=jnp.float32,
                )
                w = jnp.exp(scores * SCALE) * maskf
                wsum = jnp.sum(w, axis=-1, keepdims=True)
                ctx = jnp.dot(w.astype(BF), v_ref[:, sl],
                              preferred_element_type=jnp.float32)
                ctx_ref[:, sl] = (ctx / wsum).astype(BF)
            pbuf[rows(c), :] = jnp.dot(
                ctx_ref[:, :], wo_ref[:, :],
                preferred_element_type=jnp.float32).astype(BF)

        def cw_rdma(s):
            return pltpu.make_async_remote_copy(
                src_ref=pbuf.at[rows((pos - s) % N_DEV), CL],
                dst_ref=stag_cw.at[s],
                send_sem=send_cw.at[s], recv_sem=recv_cw.at[s],
                device_id=(right,), device_id_type=pl.DeviceIdType.MESH,
            )

        def ccw_rdma(s):
            return pltpu.make_async_remote_copy(
                src_ref=pbuf.at[rows((pos + s) % N_DEV), CR],
                dst_ref=stag_ccw.at[s],
                send_sem=send_ccw.at[s], recv_sem=recv_ccw.at[s],
                device_id=(left,), device_id_type=pl.DeviceIdType.MESH,
            )

        def accum(c, half, stag, s):
            cur = pbuf[rows(c), half].astype(jnp.float32)
            inc = stag[s].astype(jnp.float32)
            pbuf[rows(c), half] = (cur + inc).astype(BF)

        compute_chunk(pos)
        cw_rdma(0).start()
        ccw_rdma(0).start()

        compute_chunk((pos + 1) % N_DEV)
        ccw_rdma(0).wait()
        accum((pos + 1) % N_DEV, CR, stag_ccw, 0)
        ccw_rdma(1).start()

        compute_chunk((pos + 3) % N_DEV)
        cw_rdma(0).wait()
        accum((pos - 1) % N_DEV, CL, stag_cw, 0)
        cw_rdma(1).start()

        compute_chunk((pos + 2) % N_DEV)
        ccw_rdma(1).wait()
        accum((pos + 2) % N_DEV, CR, stag_ccw, 1)
        ccw_rdma(2).start()

        cw_rdma(1).wait()
        accum((pos - 2) % N_DEV, CL, stag_cw, 1)
        cw_rdma(2).start()

        ccw_rdma(2).wait()
        accum((pos + 3) % N_DEV, CR, stag_ccw, 2)
        cw_rdma(2).wait()
        accum((pos - 3) % N_DEV, CL, stag_cw, 2)

        def ag_cw(t):
            g = (pos + 1 - t) % N_DEV
            return pltpu.make_async_remote_copy(
                src_ref=pbuf.at[rows(g), CL],
                dst_ref=pbuf.at[rows(g), CL],
                send_sem=send_cw.at[3 + t], recv_sem=recv_cw.at[3 + t],
                device_id=(right,), device_id_type=pl.DeviceIdType.MESH,
            )

        def ag_ccw(t):
            g = (pos - 1 + t) % N_DEV
            return pltpu.make_async_remote_copy(
                src_ref=pbuf.at[rows(g), CR],
                dst_ref=pbuf.at[rows(g), CR],
                send_sem=send_ccw.at[3 + t], recv_sem=recv_ccw.at[3 + t],
                device_id=(left,), device_id_type=pl.DeviceIdType.MESH,
            )

        def conv(c, half):
            out_ref[rows(c), half] = pbuf[rows(c), half].astype(jnp.float32)

        ag_cw(0).start()
        ag_ccw(0).start()
        conv((pos + 1) % N_DEV, CL)
        conv((pos - 1) % N_DEV, CR)

        ag_cw(0).wait()
        ag_cw(1).start()
        conv(pos, CL)
        ag_ccw(0).wait()
        ag_ccw(1).start()
        conv(pos, CR)

        ag_cw(1).wait()
        ag_cw(2).start()
        conv((pos - 1) % N_DEV, CL)
        ag_ccw(1).wait()
        ag_ccw(2).start()
        conv((pos + 1) % N_DEV, CR)

        ag_cw(2).wait()
        conv((pos - 2) % N_DEV, CL)
        ag_ccw(2).wait()
        conv((pos + 2) % N_DEV, CR)

    out = pl.pallas_call(
        body,
        out_shape=jax.ShapeDtypeStruct((SQ, D_MODEL), jnp.float32),
        in_specs=[
            pl.BlockSpec(memory_space=pltpu.VMEM),
            pl.BlockSpec(memory_space=pltpu.VMEM),
            pl.BlockSpec(memory_space=pltpu.MemorySpace.HBM),
            pl.BlockSpec(memory_space=pltpu.MemorySpace.HBM),
            pl.BlockSpec(memory_space=pltpu.VMEM),
        ],
        out_specs=pl.BlockSpec(memory_space=pltpu.VMEM),
        scratch_shapes=[
            pltpu.VMEM((SQ // N_DEV, H_PER * DH), BF),
            pltpu.VMEM((SQ, D_MODEL), BF),
            pltpu.VMEM((3, SQ // N_DEV, D_MODEL // 2), BF),
            pltpu.VMEM((3, SQ // N_DEV, D_MODEL // 2), BF),
            pltpu.VMEM((SKV, H_PER, DH), jnp.float32),
            pltpu.VMEM((SKV, H_PER, DH), jnp.float32),
            pltpu.VMEM((SKV, H_PER * DH), BF),
            pltpu.VMEM((SKV, H_PER * DH), BF),
            pltpu.SemaphoreType.DMA((6,)),
            pltpu.SemaphoreType.DMA((6,)),
            pltpu.SemaphoreType.DMA((6,)),
            pltpu.SemaphoreType.DMA((6,)),
            pltpu.SemaphoreType.DMA((2,)),
        ],
        compiler_params=pltpu.CompilerParams(collective_id=0),
    )(x2, wq2, K_ext, V_ext, wo2)

    return out.reshape(1, SQ, D_MODEL)


# device time: 54441 ns/iter; 3.3656x vs baseline; 1.0074x over previous
import jax
import jax.numpy as jnp
from jax import lax
from jax.experimental import pallas as pl
from jax.experimental.pallas import tpu as pltpu

N_DEV = 4
SQ = 1024
SKV = 1024
D_MODEL = 1024
H_PER = 8
DH = 128
BLK = 64
SCALE = 0.08838834764831843
BF = jnp.bfloat16


def kernel(x, Wq, K_ext, V_ext, Wo):
    x2 = x[0].astype(BF)
    wq2 = Wq.astype(BF)
    wo2 = Wo.astype(BF)

    def body(x_ref, wq_ref, kext_ref, vext_ref, wo_ref, out_ref,
             ctx_ref, pbuf, stag_cw, stag_ccw, kf32, vf32, kbf, vbf,
             send_cw, recv_cw, send_ccw, recv_ccw, kv_sems):
        pos = lax.axis_index("i")
        left = (pos - 1) % N_DEV
        right = (pos + 1) % N_DEV

        kdma = pltpu.make_async_copy(
            kext_ref.at[0, :, pl.ds(pos * H_PER, H_PER), :], kf32,
            kv_sems.at[0])
        vdma = pltpu.make_async_copy(
            vext_ref.at[0, :, pl.ds(pos * H_PER, H_PER), :], vf32,
            kv_sems.at[1])
        kdma.start()
        vdma.start()

        barrier_sem = pltpu.get_barrier_semaphore()
        for nbr in [left, right]:
            pl.semaphore_signal(
                barrier_sem, inc=1,
                device_id=(nbr,), device_id_type=pl.DeviceIdType.MESH,
            )
        pl.semaphore_wait(barrier_sem, 2)

        k_ref = kbf
        v_ref = vbf

        RCH = SQ // N_DEV

        def rows(c):
            return pl.ds(c * RCH, RCH)

        CL = pl.ds(0, D_MODEL // 2)
        CR = pl.ds(D_MODEL // 2, D_MODEL // 2)

        def compute_chunk(c, q=None):
            if q is None:
                q = jnp.dot(x_ref[rows(c), :], wq_ref[:, :],
                            preferred_element_type=jnp.float32)
            qb = (lax.broadcasted_iota(jnp.int32, (RCH, SKV), 0)
                  + c * RCH) // BLK
            kb = lax.broadcasted_iota(jnp.int32, (RCH, SKV), 1) // BLK
            mask = (qb == kb) | (kb == 0) | ((qb + kb) % 3 == 0)
            maskf = mask.astype(jnp.float32)
            for h in range(H_PER):
                sl = pl.ds(h * DH, DH)
                qh = q[:, h * DH:(h + 1) * DH].astype(BF)
                kh = k_ref[:, sl]
                scores = lax.dot_general(
                    qh, kh, (((1,), (1,)), ((), ())),
                    preferred_element_type=jnp.float32,
                )
                w = jnp.exp(scores * SCALE) * maskf
                wsum = jnp.sum(w, axis=-1, keepdims=True)
                ctx = jnp.dot(w.astype(BF), v_ref[:, sl],
                              preferred_element_type=jnp.float32)
                ctx_ref[:, sl] = (ctx / wsum).astype(BF)
            pbuf[rows(c), :] = jnp.dot(
                ctx_ref[:, :], wo_ref[:, :],
                preferred_element_type=jnp.float32).astype(BF)

        def cw_rdma(s):
            return pltpu.make_async_remote_copy(
                src_ref=pbuf.at[rows((pos - s) % N_DEV), CL],
                dst_ref=stag_cw.at[s],
                send_sem=send_cw.at[s], recv_sem=recv_cw.at[s],
                device_id=(right,), device_id_type=pl.DeviceIdType.MESH,
            )

        def ccw_rdma(s):
            return pltpu.make_async_remote_copy(
                src_ref=pbuf.at[rows((pos + s) % N_DEV), CR],
                dst_ref=stag_ccw.at[s],
                send_sem=send_ccw.at[s], recv_sem=recv_ccw.at[s],
                device_id=(left,), device_id_type=pl.DeviceIdType.MESH,
            )

        def accum(c, half, stag, s):
            cur = pbuf[rows(c), half].astype(jnp.float32)
            inc = stag[s].astype(jnp.float32)
            pbuf[rows(c), half] = (cur + inc).astype(BF)

        def ag_cw(t):
            g = (pos + 1 - t) % N_DEV
            return pltpu.make_async_remote_copy(
                src_ref=pbuf.at[rows(g), CL],
                dst_ref=pbuf.at[rows(g), CL],
                send_sem=send_cw.at[3 + t], recv_sem=recv_cw.at[3 + t],
                device_id=(right,), device_id_type=pl.DeviceIdType.MESH,
            )

        def ag_ccw(t):
            g = (pos - 1 + t) % N_DEV
            return pltpu.make_async_remote_copy(
                src_ref=pbuf.at[rows(g), CR],
                dst_ref=pbuf.at[rows(g), CR],
                send_sem=send_ccw.at[3 + t], recv_sem=recv_ccw.at[3 + t],
                device_id=(left,), device_id_type=pl.DeviceIdType.MESH,
            )

        def conv(c, half):
            out_ref[rows(c), half] = pbuf[rows(c), half].astype(jnp.float32)

        q0 = jnp.dot(x_ref[rows(pos), :], wq_ref[:, :],
                     preferred_element_type=jnp.float32)
        kdma.wait()
        kbf[:, :] = kf32[:, :, :].reshape(SKV, H_PER * DH).astype(BF)
        vdma.wait()
        vbf[:, :] = vf32[:, :, :].reshape(SKV, H_PER * DH).astype(BF)

        compute_chunk(pos, q0)
        cw_rdma(0).start()
        ccw_rdma(0).start()

        compute_chunk((pos + 1) % N_DEV)
        ccw_rdma(0).wait_recv()
        accum((pos + 1) % N_DEV, CR, stag_ccw, 0)
        ccw_rdma(1).start()

        compute_chunk((pos + 3) % N_DEV)
        cw_rdma(0).wait_recv()
        accum((pos - 1) % N_DEV, CL, stag_cw, 0)
        cw_rdma(1).start()

        compute_chunk((pos + 2) % N_DEV)
        ccw_rdma(1).wait_recv()
        accum((pos + 2) % N_DEV, CR, stag_ccw, 1)
        ccw_rdma(2).start()

        cw_rdma(1).wait_recv()
        accum((pos - 2) % N_DEV, CL, stag_cw, 1)
        cw_rdma(2).start()

        ccw_rdma(2).wait_recv()
        accum((pos + 3) % N_DEV, CR, stag_ccw, 2)
        ag_ccw(0).start()
        cw_rdma(2).wait_recv()
        accum((pos - 3) % N_DEV, CL, stag_cw, 2)
        ag_cw(0).start()

        conv((pos + 1) % N_DEV, CL)
        conv((pos - 1) % N_DEV, CR)

        ag_cw(0).wait_recv()
        ag_cw(1).start()
        conv(pos, CL)
        ag_ccw(0).wait_recv()
        ag_ccw(1).start()
        conv(pos, CR)

        ag_cw(1).wait_recv()
        ag_cw(2).start()
        conv((pos - 1) % N_DEV, CL)
        ag_ccw(1).wait_recv()
        ag_ccw(2).start()
        conv((pos + 1) % N_DEV, CR)

        ag_cw(2).wait_recv()
        conv((pos - 2) % N_DEV, CL)
        ag_ccw(2).wait_recv()
        conv((pos + 2) % N_DEV, CR)

        for s in range(3):
            cw_rdma(s).wait_send()
            ccw_rdma(s).wait_send()
        for t in range(3):
            ag_cw(t).wait_send()
            ag_ccw(t).wait_send()

    out = pl.pallas_call(
        body,
        out_shape=jax.ShapeDtypeStruct((SQ, D_MODEL), jnp.float32),
        in_specs=[
            pl.BlockSpec(memory_space=pltpu.VMEM),
            pl.BlockSpec(memory_space=pltpu.VMEM),
            pl.BlockSpec(memory_space=pltpu.MemorySpace.HBM),
            pl.BlockSpec(memory_space=pltpu.MemorySpace.HBM),
            pl.BlockSpec(memory_space=pltpu.VMEM),
        ],
        out_specs=pl.BlockSpec(memory_space=pltpu.VMEM),
        scratch_shapes=[
            pltpu.VMEM((SQ // N_DEV, H_PER * DH), BF),
            pltpu.VMEM((SQ, D_MODEL), BF),
            pltpu.VMEM((3, SQ // N_DEV, D_MODEL // 2), BF),
            pltpu.VMEM((3, SQ // N_DEV, D_MODEL // 2), BF),
            pltpu.VMEM((SKV, H_PER, DH), jnp.float32),
            pltpu.VMEM((SKV, H_PER, DH), jnp.float32),
            pltpu.VMEM((SKV, H_PER * DH), BF),
            pltpu.VMEM((SKV, H_PER * DH), BF),
            pltpu.SemaphoreType.DMA((6,)),
            pltpu.SemaphoreType.DMA((6,)),
            pltpu.SemaphoreType.DMA((6,)),
            pltpu.SemaphoreType.DMA((6,)),
            pltpu.SemaphoreType.DMA((2,)),
        ],
        compiler_params=pltpu.CompilerParams(collective_id=0),
    )(x2, wq2, K_ext, V_ext, wo2)

    return out.reshape(1, SQ, D_MODEL)


# device time: 53595 ns/iter; 3.4187x vs baseline; 1.0158x over previous
import jax
import jax.numpy as jnp
from jax import lax
from jax.experimental import pallas as pl
from jax.experimental.pallas import tpu as pltpu

N_DEV = 4
SQ = 1024
SKV = 1024
D_MODEL = 1024
H_PER = 8
DH = 128
BLK = 64
SCALE = 0.08838834764831843
BF = jnp.bfloat16


def kernel(x, Wq, K_ext, V_ext, Wo):
    x2 = x[0].astype(BF)
    wq2 = Wq.astype(BF)
    wo2 = Wo.astype(BF)

    def body(x_ref, wq_ref, kext_ref, vext_ref, wo_ref, out_ref,
             ctx_ref, pbuf, stag_cw, stag_ccw, kf32, vf32, kbf, vbf,
             send_cw, recv_cw, send_ccw, recv_ccw, kv_sems):
        pos = lax.axis_index("i")
        left = (pos - 1) % N_DEV
        right = (pos + 1) % N_DEV

        kdma = pltpu.make_async_copy(
            kext_ref.at[0, :, pl.ds(pos * H_PER, H_PER), :], kf32,
            kv_sems.at[0])
        vdma = pltpu.make_async_copy(
            vext_ref.at[0, :, pl.ds(pos * H_PER, H_PER), :], vf32,
            kv_sems.at[1])
        kdma.start()
        vdma.start()

        barrier_sem = pltpu.get_barrier_semaphore()
        for nbr in [left, right]:
            pl.semaphore_signal(
                barrier_sem, inc=1,
                device_id=(nbr,), device_id_type=pl.DeviceIdType.MESH,
            )
        pl.semaphore_wait(barrier_sem, 2)

        k_ref = kbf
        v_ref = vbf

        RCH = SQ // N_DEV

        def rows(c):
            return pl.ds(c * RCH, RCH)

        CL = pl.ds(0, D_MODEL // 2)
        CR = pl.ds(D_MODEL // 2, D_MODEL // 2)

        def compute_chunk(c, q=None):
            if q is None:
                q = jnp.dot(x_ref[rows(c), :], wq_ref[:, :],
                            preferred_element_type=jnp.float32)
            qb = (lax.broadcasted_iota(jnp.int32, (RCH, SKV), 0)
                  + c * RCH) // BLK
            kb = lax.broadcasted_iota(jnp.int32, (RCH, SKV), 1) // BLK
            mask = (qb == kb) | (kb == 0) | ((qb + kb) % 3 == 0)
            maskf = mask.astype(jnp.float32)
            for h in range(H_PER):
                sl = pl.ds(h * DH, DH)
                qh = q[:, h * DH:(h + 1) * DH].astype(BF)
                kh = k_ref[:, sl]
                scores = lax.dot_general(
                    qh, kh, (((1,), (1,)), ((), ())),
                    preferred_element_type=jnp.float32,
                )
                w = jnp.exp(scores * SCALE) * maskf
                wsum = jnp.sum(w, axis=-1, keepdims=True)
                ctx = jnp.dot(w.astype(BF), v_ref[:, sl],
                              preferred_element_type=jnp.float32)
                ctx_ref[:, sl] = (ctx / wsum).astype(BF)
            pbuf[rows(c), :] = jnp.dot(
                ctx_ref[:, :], wo_ref[:, :],
                preferred_element_type=jnp.float32).astype(BF)

        def cw_rdma(s):
            return pltpu.make_async_remote_copy(
                src_ref=pbuf.at[rows((pos - s) % N_DEV), CL],
                dst_ref=stag_cw.at[s],
                send_sem=send_cw.at[s], recv_sem=recv_cw.at[s],
                device_id=(right,), device_id_type=pl.DeviceIdType.MESH,
            )

        def ccw_rdma(s):
            return pltpu.make_async_remote_copy(
                src_ref=pbuf.at[rows((pos + s) % N_DEV), CR],
                dst_ref=stag_ccw.at[s],
                send_sem=send_ccw.at[s], recv_sem=recv_ccw.at[s],
                device_id=(left,), device_id_type=pl.DeviceIdType.MESH,
            )

        def accum(c, half, stag, s):
            cur = pbuf[rows(c), half].astype(jnp.float32)
            inc = stag[s].astype(jnp.float32)
            pbuf[rows(c), half] = (cur + inc).astype(BF)

        def ag_cw(d):
            return pltpu.make_async_remote_copy(
                src_ref=pbuf.at[rows((pos + 1) % N_DEV), CL],
                dst_ref=pbuf.at[rows((pos + 1) % N_DEV), CL],
                send_sem=send_cw.at[2 + d], recv_sem=recv_cw.at[2 + d],
                device_id=((pos + d) % N_DEV,),
                device_id_type=pl.DeviceIdType.MESH,
            )

        def ag_ccw(d):
            return pltpu.make_async_remote_copy(
                src_ref=pbuf.at[rows((pos - 1) % N_DEV), CR],
                dst_ref=pbuf.at[rows((pos - 1) % N_DEV), CR],
                send_sem=send_ccw.at[2 + d], recv_sem=recv_ccw.at[2 + d],
                device_id=((pos + d) % N_DEV,),
                device_id_type=pl.DeviceIdType.MESH,
            )

        def conv(c, half):
            out_ref[rows(c), half] = pbuf[rows(c), half].astype(jnp.float32)

        q0 = jnp.dot(x_ref[rows(pos), :], wq_ref[:, :],
                     preferred_element_type=jnp.float32)
        kdma.wait()
        kbf[:, :] = kf32[:, :, :].reshape(SKV, H_PER * DH).astype(BF)
        vdma.wait()
        vbf[:, :] = vf32[:, :, :].reshape(SKV, H_PER * DH).astype(BF)

        compute_chunk(pos, q0)
        cw_rdma(0).start()
        ccw_rdma(0).start()

        compute_chunk((pos + 1) % N_DEV)
        ccw_rdma(0).wait_recv()
        accum((pos + 1) % N_DEV, CR, stag_ccw, 0)
        ccw_rdma(1).start()

        compute_chunk((pos + 3) % N_DEV)
        cw_rdma(0).wait_recv()
        accum((pos - 1) % N_DEV, CL, stag_cw, 0)
        cw_rdma(1).start()

        compute_chunk((pos + 2) % N_DEV)
        ccw_rdma(1).wait_recv()
        accum((pos + 2) % N_DEV, CR, stag_ccw, 1)
        ccw_rdma(2).start()

        cw_rdma(1).wait_recv()
        accum((pos - 2) % N_DEV, CL, stag_cw, 1)
        cw_rdma(2).start()

        ccw_rdma(2).wait_recv()
        accum((pos + 3) % N_DEV, CR, stag_ccw, 2)
        ag_ccw(1).start()
        ag_ccw(2).start()
        ag_ccw(3).start()
        cw_rdma(2).wait_recv()
        accum((pos - 3) % N_DEV, CL, stag_cw, 2)
        ag_cw(1).start()
        ag_cw(2).start()
        ag_cw(3).start()

        conv((pos + 1) % N_DEV, CL)
        conv((pos - 1) % N_DEV, CR)

        for d in (1, 2, 3):
            ag_cw(d).wait_recv()
            conv((pos - d + 1) % N_DEV, CL)
            ag_ccw(d).wait_recv()
            conv((pos - d - 1) % N_DEV, CR)

        for s in range(3):
            cw_rdma(s).wait_send()
            ccw_rdma(s).wait_send()
        for d in (1, 2, 3):
            ag_cw(d).wait_send()
            ag_ccw(d).wait_send()

    out = pl.pallas_call(
        body,
        out_shape=jax.ShapeDtypeStruct((SQ, D_MODEL), jnp.float32),
        in_specs=[
            pl.BlockSpec(memory_space=pltpu.VMEM),
            pl.BlockSpec(memory_space=pltpu.VMEM),
            pl.BlockSpec(memory_space=pltpu.MemorySpace.HBM),
            pl.BlockSpec(memory_space=pltpu.MemorySpace.HBM),
            pl.BlockSpec(memory_space=pltpu.VMEM),
        ],
        out_specs=pl.BlockSpec(memory_space=pltpu.VMEM),
        scratch_shapes=[
            pltpu.VMEM((SQ // N_DEV, H_PER * DH), BF),
            pltpu.VMEM((SQ, D_MODEL), BF),
            pltpu.VMEM((3, SQ // N_DEV, D_MODEL // 2), BF),
            pltpu.VMEM((3, SQ // N_DEV, D_MODEL // 2), BF),
            pltpu.VMEM((SKV, H_PER, DH), jnp.float32),
            pltpu.VMEM((SKV, H_PER, DH), jnp.float32),
            pltpu.VMEM((SKV, H_PER * DH), BF),
            pltpu.VMEM((SKV, H_PER * DH), BF),
            pltpu.SemaphoreType.DMA((6,)),
            pltpu.SemaphoreType.DMA((6,)),
            pltpu.SemaphoreType.DMA((6,)),
            pltpu.SemaphoreType.DMA((6,)),
            pltpu.SemaphoreType.DMA((2,)),
        ],
        compiler_params=pltpu.CompilerParams(collective_id=0),
    )(x2, wq2, K_ext, V_ext, wo2)

    return out.reshape(1, SQ, D_MODEL)


# device time: 52924 ns/iter; 3.4621x vs baseline; 1.0127x over previous
import jax
import jax.numpy as jnp
from jax import lax
from jax.experimental import pallas as pl
from jax.experimental.pallas import tpu as pltpu

N_DEV = 4
SQ = 1024
SKV = 1024
D_MODEL = 1024
H_PER = 8
DH = 128
BLK = 64
SCALE = 0.08838834764831843
BF = jnp.bfloat16


def kernel(x, Wq, K_ext, V_ext, Wo):
    x2 = x[0].astype(BF)
    wq2 = Wq.astype(BF)
    wo2 = Wo.astype(BF)

    def body(x_ref, wq_ref, kext_ref, vext_ref, wo_ref, out_ref,
             ctx_ref, pbuf, stag_cw, stag_ccw, kf32, vf32, kbf, vbf,
             send_cw, recv_cw, send_ccw, recv_ccw, kv_sems):
        pos = lax.axis_index("i")
        left = (pos - 1) % N_DEV
        right = (pos + 1) % N_DEV

        kdma = pltpu.make_async_copy(
            kext_ref.at[0, :, pl.ds(pos * H_PER, H_PER), :], kf32,
            kv_sems.at[0])
        vdma = pltpu.make_async_copy(
            vext_ref.at[0, :, pl.ds(pos * H_PER, H_PER), :], vf32,
            kv_sems.at[1])
        kdma.start()
        vdma.start()

        barrier_sem = pltpu.get_barrier_semaphore()
        for nbr in [left, right]:
            pl.semaphore_signal(
                barrier_sem, inc=1,
                device_id=(nbr,), device_id_type=pl.DeviceIdType.MESH,
            )

        k_ref = kbf
        v_ref = vbf

        RCH = SQ // N_DEV

        def rows(c):
            return pl.ds(c * RCH, RCH)

        CL = pl.ds(0, D_MODEL // 2)
        CR = pl.ds(D_MODEL // 2, D_MODEL // 2)

        def compute_chunk(c, q=None):
            if q is None:
                q = jnp.dot(x_ref[rows(c), :], wq_ref[:, :],
                            preferred_element_type=jnp.float32)
            qb = (lax.broadcasted_iota(jnp.int32, (RCH, SKV), 0)
                  + c * RCH) // BLK
            kb = lax.broadcasted_iota(jnp.int32, (RCH, SKV), 1) // BLK
            mask = (qb == kb) | (kb == 0) | ((qb + kb) % 3 == 0)
            maskf = mask.astype(jnp.float32)
            for h in range(H_PER):
                sl = pl.ds(h * DH, DH)
                qh = q[:, h * DH:(h + 1) * DH].astype(BF)
                kh = k_ref[:, sl]
                scores = lax.dot_general(
                    qh, kh, (((1,), (1,)), ((), ())),
                    preferred_element_type=jnp.float32,
                )
                w = jnp.exp(scores * SCALE) * maskf
                wsum = jnp.sum(w, axis=-1, keepdims=True)
                ctx = jnp.dot(w.astype(BF), v_ref[:, sl],
                              preferred_element_type=jnp.float32)
                ctx_ref[:, sl] = (ctx / wsum).astype(BF)
            pbuf[rows(c), :] = jnp.dot(
                ctx_ref[:, :], wo_ref[:, :],
                preferred_element_type=jnp.float32).astype(BF)

        def cw_rdma(s):
            return pltpu.make_async_remote_copy(
                src_ref=pbuf.at[rows((pos - s) % N_DEV), CL],
                dst_ref=stag_cw.at[s],
                send_sem=send_cw.at[s], recv_sem=recv_cw.at[s],
                device_id=(right,), device_id_type=pl.DeviceIdType.MESH,
            )

        def ccw_rdma(s):
            return pltpu.make_async_remote_copy(
                src_ref=pbuf.at[rows((pos + s) % N_DEV), CR],
                dst_ref=stag_ccw.at[s],
                send_sem=send_ccw.at[s], recv_sem=recv_ccw.at[s],
                device_id=(left,), device_id_type=pl.DeviceIdType.MESH,
            )

        def accum(c, half, stag, s):
            cur = pbuf[rows(c), half].astype(jnp.float32)
            inc = stag[s].astype(jnp.float32)
            pbuf[rows(c), half] = (cur + inc).astype(BF)

        def ag_cw(d):
            return pltpu.make_async_remote_copy(
                src_ref=pbuf.at[rows((pos + 1) % N_DEV), CL],
                dst_ref=pbuf.at[rows((pos + 1) % N_DEV), CL],
                send_sem=send_cw.at[2 + d], recv_sem=recv_cw.at[2 + d],
                device_id=((pos + d) % N_DEV,),
                device_id_type=pl.DeviceIdType.MESH,
            )

        def ag_ccw(d):
            return pltpu.make_async_remote_copy(
                src_ref=pbuf.at[rows((pos - 1) % N_DEV), CR],
                dst_ref=pbuf.at[rows((pos - 1) % N_DEV), CR],
                send_sem=send_ccw.at[2 + d], recv_sem=recv_ccw.at[2 + d],
                device_id=((pos + d) % N_DEV,),
                device_id_type=pl.DeviceIdType.MESH,
            )

        def conv(c, half):
            out_ref[rows(c), half] = pbuf[rows(c), half].astype(jnp.float32)

        q0 = jnp.dot(x_ref[rows(pos), :], wq_ref[:, :],
                     preferred_element_type=jnp.float32)
        kdma.wait()
        kbf[:, :] = kf32[:, :, :].reshape(SKV, H_PER * DH).astype(BF)
        vdma.wait()
        vbf[:, :] = vf32[:, :, :].reshape(SKV, H_PER * DH).astype(BF)

        compute_chunk(pos, q0)
        pl.semaphore_wait(barrier_sem, 2)
        cw_rdma(0).start()
        ccw_rdma(0).start()

        compute_chunk((pos + 1) % N_DEV)
        ccw_rdma(0).wait_recv()
        accum((pos + 1) % N_DEV, CR, stag_ccw, 0)
        ccw_rdma(1).start()

        compute_chunk((pos + 3) % N_DEV)
        cw_rdma(0).wait_recv()
        accum((pos - 1) % N_DEV, CL, stag_cw, 0)
        cw_rdma(1).start()

        compute_chunk((pos + 2) % N_DEV)
        ccw_rdma(1).wait_recv()
        accum((pos + 2) % N_DEV, CR, stag_ccw, 1)
        ccw_rdma(2).start()

        cw_rdma(1).wait_recv()
        accum((pos - 2) % N_DEV, CL, stag_cw, 1)
        cw_rdma(2).start()

        ccw_rdma(2).wait_recv()
        accum((pos + 3) % N_DEV, CR, stag_ccw, 2)
        ag_ccw(1).start()
        ag_ccw(2).start()
        ag_ccw(3).start()
        cw_rdma(2).wait_recv()
        accum((pos - 3) % N_DEV, CL, stag_cw, 2)
        ag_cw(1).start()
        ag_cw(2).start()
        ag_cw(3).start()

        conv((pos + 1) % N_DEV, CL)
        conv((pos - 1) % N_DEV, CR)

        for d in (1, 2, 3):
            ag_cw(d).wait_recv()
            conv((pos - d + 1) % N_DEV, CL)
            ag_ccw(d).wait_recv()
            conv((pos - d - 1) % N_DEV, CR)

        for s in range(3):
            cw_rdma(s).wait_send()
            ccw_rdma(s).wait_send()
        for d in (1, 2, 3):
            ag_cw(d).wait_send()
            ag_ccw(d).wait_send()

    out = pl.pallas_call(
        body,
        out_shape=jax.ShapeDtypeStruct((SQ, D_MODEL), jnp.float32),
        in_specs=[
            pl.BlockSpec(memory_space=pltpu.VMEM),
            pl.BlockSpec(memory_space=pltpu.VMEM),
            pl.BlockSpec(memory_space=pltpu.MemorySpace.HBM),
            pl.BlockSpec(memory_space=pltpu.MemorySpace.HBM),
            pl.BlockSpec(memory_space=pltpu.VMEM),
        ],
        out_specs=pl.BlockSpec(memory_space=pltpu.VMEM),
        scratch_shapes=[
            pltpu.VMEM((SQ // N_DEV, H_PER * DH), BF),
            pltpu.VMEM((SQ, D_MODEL), BF),
            pltpu.VMEM((3, SQ // N_DEV, D_MODEL // 2), BF),
            pltpu.VMEM((3, SQ // N_DEV, D_MODEL // 2), BF),
            pltpu.VMEM((SKV, H_PER, DH), jnp.float32),
            pltpu.VMEM((SKV, H_PER, DH), jnp.float32),
            pltpu.VMEM((SKV, H_PER * DH), BF),
            pltpu.VMEM((SKV, H_PER * DH), BF),
            pltpu.SemaphoreType.DMA((6,)),
            pltpu.SemaphoreType.DMA((6,)),
            pltpu.SemaphoreType.DMA((6,)),
            pltpu.SemaphoreType.DMA((6,)),
            pltpu.SemaphoreType.DMA((2,)),
        ],
        compiler_params=pltpu.CompilerParams(collective_id=0),
    )(x2, wq2, K_ext, V_ext, wo2)

    return out.reshape(1, SQ, D_MODEL)
